# Initial kernel scaffold; baseline (speedup 1.0000x reference)
import jax
import jax.numpy as jnp
from jax.experimental import pallas as pl


def kernel(x, edge_index, W1, b1, W2, b2):
    raise NotImplementedError("write your pallas kernel here")


# speedup vs baseline: 13.2526x; 13.2526x over previous
import functools

import jax
import jax.numpy as jnp
from jax import lax
from jax.experimental import pallas as pl
from jax.experimental.pallas import tpu as pltpu
from jax.experimental.pallas import tpu_sc as plsc

NC = 2
NS = 16
NW = NC * NS
LANES = 16
CHUNK = 128

def _mesh():
    return plsc.VectorSubcoreMesh(
        core_axis_name="c", subcore_axis_name="s", num_cores=NC, num_subcores=NS
    )


def _acc_rows(n):
    blk = NS * 128
    return ((n + 1 + blk - 1) // blk) * blk


def _sc_degree(dst3, n):
    C = dst3.shape[1]
    R = _acc_rows(n)
    rps = R // NS

    @functools.partial(
        pl.kernel,
        out_type=jax.ShapeDtypeStruct((NC, R, LANES), jnp.float32),
        mesh=_mesh(),
        scratch_types=[
            pltpu.VMEM((C, CHUNK), jnp.int32),
            pltpu.VMEM((128, LANES), jnp.float32),
            pltpu.VMEM_SHARED((R, LANES), jnp.float32),
        ],
    )
    def k(dst_hbm, out_hbm, dst_v, fill_v, acc):
        cid = lax.axis_index("c")
        sid = lax.axis_index("s")
        wid = sid * NC + cid
        pltpu.sync_copy(dst_hbm.at[wid], dst_v)

        zero = jnp.zeros((LANES,), jnp.float32)

        @pl.loop(0, 128)
        def _(r):
            fill_v[r, pl.ds(0, LANES)] = zero

        base = sid * rps

        @pl.loop(0, rps, step=128)
        def _(r):
            pltpu.sync_copy(fill_v, acc.at[pl.ds(base + r, 128)])

        one = jnp.ones((LANES,), jnp.float32)

        @pl.loop(0, 128)
        def _(r):
            fill_v[r, pl.ds(0, LANES)] = one

        plsc.subcore_barrier()

        @pl.loop(0, C)
        def _(j):
            pltpu.sync_copy(fill_v, acc.at[dst_v.at[j]], add=True)

        plsc.subcore_barrier()
        pltpu.sync_copy(
            acc.at[pl.ds(base, rps)], out_hbm.at[cid].at[pl.ds(base, rps)]
        )

    return k(dst3)


def _sc_scatter(h, src3, dst3):
    n, D = h.shape
    C = src3.shape[1]
    R = _acc_rows(n)
    rps = R // NS

    @functools.partial(
        pl.kernel,
        out_type=jax.ShapeDtypeStruct((NC, R, D), jnp.float32),
        mesh=_mesh(),
        scratch_types=[
            pltpu.VMEM((C, CHUNK), jnp.int32),
            pltpu.VMEM((C, CHUNK), jnp.int32),
            pltpu.VMEM((CHUNK,), jnp.int32),
            pltpu.VMEM((CHUNK, D), jnp.float32),
            pltpu.VMEM_SHARED((R, D), jnp.float32),
            pltpu.SemaphoreType.DMA,
            pltpu.SemaphoreType.DMA,
        ],
    )
    def k(h_hbm, src_hbm, dst_hbm, out_hbm, src_v, dst_v, src_cur, buf, acc,
          gsem, ssem):
        cid = lax.axis_index("c")
        sid = lax.axis_index("s")
        wid = sid * NC + cid
        pltpu.sync_copy(src_hbm.at[wid], src_v)
        pltpu.sync_copy(dst_hbm.at[wid], dst_v)

        zero = jnp.zeros((LANES,), jnp.float32)

        @pl.loop(0, CHUNK)
        def _(r):
            for cix in range(0, D, LANES):
                buf[r, pl.ds(cix, LANES)] = zero

        base = sid * rps

        @pl.loop(0, rps, step=CHUNK)
        def _(r):
            pltpu.sync_copy(buf, acc.at[pl.ds(base + r, CHUNK)])

        plsc.subcore_barrier()

        @pl.loop(0, C)
        def _(j):
            @pl.loop(0, CHUNK, step=LANES)
            def _(t):
                src_cur[pl.ds(t, LANES)] = src_v[j, pl.ds(t, LANES)]

            pltpu.async_copy(h_hbm.at[src_cur], buf, gsem).wait()
            pltpu.async_copy(buf, acc.at[dst_v.at[j]], ssem, add=True).wait()

        plsc.subcore_barrier()
        pltpu.sync_copy(
            acc.at[pl.ds(base, rps)], out_hbm.at[cid].at[pl.ds(base, rps)]
        )

    return k(h, src3, dst3)


def _sc_gather_test(h, src3):
    n, D = h.shape
    C = src3.shape[1]

    @functools.partial(
        pl.kernel,
        out_type=jax.ShapeDtypeStruct((NW, C * CHUNK, D), jnp.float32),
        mesh=_mesh(),
        scratch_types=[
            pltpu.VMEM((C, CHUNK), jnp.int32),
            pltpu.VMEM((CHUNK,), jnp.int32),
            pltpu.VMEM((CHUNK, D), jnp.float32),
        ],
    )
    def k(h_hbm, src_hbm, out_hbm, src_v, src_cur, buf):
        cid = lax.axis_index("c")
        sid = lax.axis_index("s")
        wid = sid * NC + cid
        pltpu.sync_copy(src_hbm.at[wid], src_v)

        @pl.loop(0, C)
        def _(j):
            @pl.loop(0, CHUNK, step=LANES)
            def _(t):
                src_cur[pl.ds(t, LANES)] = src_v[j, pl.ds(t, LANES)]

            pltpu.sync_copy(h_hbm.at[src_cur], buf)
            pltpu.sync_copy(buf, out_hbm.at[wid].at[pl.ds(j * CHUNK, CHUNK)])

    return k(h, src3)


def _tc_matmul(x, W):
    def body(x_ref, w_ref, o_ref):
        o_ref[...] = jnp.dot(
            x_ref[...], w_ref[...], preferred_element_type=jnp.float32
        )

    return pl.pallas_call(
        body,
        out_shape=jax.ShapeDtypeStruct((x.shape[0], W.shape[1]), jnp.float32),
    )(x, W)


def _tc_prep(xw, degp):
    n = xw.shape[0]

    def body(xw_ref, degp_ref, h_ref, dinv_ref):
        deg = degp_ref[0, :n, 0:1] + degp_ref[1, :n, 0:1] + 1.0
        dinv = lax.rsqrt(deg)
        dinv_ref[...] = dinv
        h_ref[...] = xw_ref[...] * dinv

    return pl.pallas_call(
        body,
        out_shape=(
            jax.ShapeDtypeStruct(xw.shape, jnp.float32),
            jax.ShapeDtypeStruct((n, 1), jnp.float32),
        ),
    )(xw, degp)


def _elu(x):
    return jnp.where(x > 0, x, jnp.exp(jnp.minimum(x, 0.0)) - 1.0)


def _tc_mid(acc, hp, dinv, b, W2):
    n, D = hp.shape

    def body(a_ref, h_ref, d_ref, b_ref, w_ref, o_ref):
        s = a_ref[0, :n, :] + a_ref[1, :n, :] + h_ref[...]
        act = _elu(s * d_ref[...] + b_ref[...])
        o_ref[...] = (
            jnp.dot(act, w_ref[...], preferred_element_type=jnp.float32)
            * d_ref[...]
        )

    return pl.pallas_call(
        body, out_shape=jax.ShapeDtypeStruct((n, D), jnp.float32)
    )(acc, hp, dinv, b.reshape(1, D), W2)


def _tc_final(acc, hp, dinv, b):
    n, D = hp.shape

    def body(a_ref, h_ref, d_ref, b_ref, o_ref):
        s = a_ref[0, :n, :] + a_ref[1, :n, :] + h_ref[...]
        o_ref[...] = _elu(s * d_ref[...] + b_ref[...])

    return pl.pallas_call(
        body, out_shape=jax.ShapeDtypeStruct((n, D), jnp.float32)
    )(acc, hp, dinv, b.reshape(1, D))


def kernel(x, edge_index, W1, b1, W2, b2):
    n, D = x.shape
    E = edge_index.shape[1]

    slots_per_tile = -(-E // (NW * CHUNK)) * CHUNK
    pad = NW * slots_per_tile - E
    src = edge_index[0].astype(jnp.int32)
    dst = edge_index[1].astype(jnp.int32)
    src3 = jnp.concatenate([src, jnp.zeros((pad,), jnp.int32)]).reshape(
        NW, slots_per_tile // CHUNK, CHUNK
    )
    dst3 = jnp.concatenate([dst, jnp.full((pad,), n, jnp.int32)]).reshape(
        NW, slots_per_tile // CHUNK, CHUNK
    )

    R = _acc_rows(n)

    def _jnp_scatter(h, s, d):
        a = jnp.zeros((R, h.shape[1]), jnp.float32).at[d].add(h[s])
        return jnp.stack([a, jnp.zeros_like(a)])

    degp = _sc_degree(dst3, n)
    xw1 = _tc_matmul(x, W1)
    h1p, dinv = _tc_prep(xw1, degp)
    acc1 = _sc_scatter(h1p, src3, dst3)
    h2p = _tc_mid(acc1, h1p, dinv, b1, W2)
    acc2 = _sc_scatter(h2p, src3, dst3)
    return _tc_final(acc2, h2p, dinv, b2)
